# speedup vs baseline: 22.4770x; 22.4770x over previous
import numpy as np
import jax
import jax.numpy as jnp
from jax.experimental import pallas as pl

_DIM = 256
_G = 16
_DPG = _DIM // _G
_ROWS_PER_BLOCK = 2048


def _pack_matrix() -> np.ndarray:
    p = np.zeros((_G, _DIM), dtype=np.float32)
    for g in range(_G):
        for j in range(_DPG):
            p[g, g * _DPG + j] = float(2 ** (_DPG - 1 - j))
    return p


def _bsq_body(x_ref, p_ref, q_ref, idx_ref):
    x = x_ref[...]
    pos = x > 0
    q_ref[...] = jnp.where(pos, jnp.float32(0.25), jnp.float32(-0.25))
    bits = pos.astype(jnp.float32)
    packed = jax.lax.dot_general(
        p_ref[...], bits, (((1,), (1,)), ((), ())),
        preferred_element_type=jnp.float32)
    idx_ref[...] = packed.astype(jnp.int32)


def kernel(x):
    b, n, dim = x.shape
    rows = b * n
    xf = x.reshape(rows, dim)
    rpb = _ROWS_PER_BLOCK
    grid = (rows // rpb,)
    q, idx = pl.pallas_call(
        _bsq_body,
        grid=grid,
        in_specs=[
            pl.BlockSpec((rpb, dim), lambda i: (i, 0)),
            pl.BlockSpec((_G, dim), lambda i: (0, 0)),
        ],
        out_specs=[
            pl.BlockSpec((rpb, dim), lambda i: (i, 0)),
            pl.BlockSpec((_G, rpb), lambda i: (0, i)),
        ],
        out_shape=[
            jax.ShapeDtypeStruct((rows, dim), jnp.float32),
            jax.ShapeDtypeStruct((_G, rows), jnp.int32),
        ],
    )(xf, jnp.asarray(_pack_matrix()))
    quantized = q.reshape(b, n, dim)
    all_indices = idx.reshape(_G, b, n)
    aux_losses = jnp.zeros((_G,), dtype=jnp.float32)
    return (quantized, all_indices, aux_losses)
